# baseline (speedup 1.0000x reference)
import functools

import jax
import jax.numpy as jnp
from jax import lax
from jax.experimental import pallas as pl
from jax.experimental.pallas import tpu as pltpu
from jax.experimental.pallas import tpu_sc as plsc

B = 16384
D_IN = 64
D_HID = 128
D_OUT = 64
BLKC = 8192
CHUNK = 4096
L = 16
UNROLL = 16


@functools.lru_cache(maxsize=None)
def _make_gather(vocab):
    info = plsc.get_sparse_core_info()
    nc, ns = info.num_cores, info.num_subcores
    nw = nc * ns
    dims_per_w = D_IN // nw
    n_chunks = B // CHUNK

    mesh = plsc.VectorSubcoreMesh(core_axis_name="c", subcore_axis_name="s")

    @functools.partial(
        pl.kernel,
        mesh=mesh,
        out_type=jax.ShapeDtypeStruct((D_IN, B), jnp.float32),
        scratch_types=[
            pltpu.VMEM((B,), jnp.int32),
            pltpu.VMEM((vocab,), jnp.float32),
            pltpu.VMEM((CHUNK,), jnp.float32),
            pltpu.VMEM((CHUNK,), jnp.float32),
            pltpu.SemaphoreType.DMA,
            pltpu.SemaphoreType.DMA,
            pltpu.SemaphoreType.DMA,
            pltpu.SemaphoreType.DMA,
        ],
        compiler_params=pltpu.CompilerParams(needs_layout_passes=False),
    )
    def gather(table_t_hbm, idx_hbm, out_hbm, idx_v, col_v, out_a, out_b,
               sem_i, sem_c, sem_a, sem_b):
        wid = lax.axis_index("s") * nc + lax.axis_index("c")
        d0 = wid * dims_per_w
        idx_cp = pltpu.make_async_copy(idx_hbm, idx_v, sem_i)
        idx_cp.start()
        col_cp = pltpu.make_async_copy(table_t_hbm.at[d0], col_v, sem_c)
        col_cp.start()
        idx_cp.wait()
        col_cp.wait()
        outs = (out_a, out_b)
        sems = (sem_a, sem_b)
        for r in range(dims_per_w):
            d = d0 + r
            for chunk in range(n_chunks):
                out_v = outs[chunk % 2]
                sem_o = sems[chunk % 2]
                if r * n_chunks + chunk >= 2:
                    pltpu.make_async_copy(
                        out_v,
                        out_hbm.at[d0 + (r * n_chunks + chunk - 2) // n_chunks,
                                   pl.ds(((r * n_chunks + chunk - 2) % n_chunks)
                                         * CHUNK, CHUNK)],
                        sem_o).wait()

                @plsc.parallel_loop(0, CHUNK, step=L, unroll=UNROLL)
                def _(i, chunk=chunk, out_v=out_v):
                    iv = idx_v[pl.ds(chunk * CHUNK + i, L)]
                    out_v[pl.ds(i, L)] = plsc.load_gather(col_v, [iv])

                if r == 0 and chunk == n_chunks - 1 and dims_per_w > 1:
                    col_cp2 = pltpu.make_async_copy(
                        table_t_hbm.at[d0 + 1], col_v, sem_c)
                    col_cp2.start()
                pltpu.make_async_copy(
                    out_v, out_hbm.at[d, pl.ds(chunk * CHUNK, CHUNK)],
                    sem_o).start()
            if r == 0 and dims_per_w > 1:
                pltpu.make_async_copy(
                    table_t_hbm.at[d0 + 1], col_v, sem_c).wait()
        for chunk in (n_chunks - 2, n_chunks - 1):
            pltpu.make_async_copy(
                outs[chunk % 2],
                out_hbm.at[d0 + dims_per_w - 1, pl.ds(chunk * CHUNK, CHUNK)],
                sems[chunk % 2]).wait()

    return gather


def _mlp_body(x_ref, wc_ref, bc_ref, w1_ref, b1_ref, w2_ref, b2_ref, o_ref):
    dn = (((1,), (0,)), ((), ()))
    x = x_ref[...]
    c = lax.dot_general(wc_ref[...], x, dn,
                        preferred_element_type=jnp.float32) + bc_ref[...]
    h = jnp.maximum(
        lax.dot_general(w1_ref[...], c, dn,
                        preferred_element_type=jnp.float32) + b1_ref[...], 0.0)
    o_ref[...] = lax.dot_general(w2_ref[...], h, dn,
                                 preferred_element_type=jnp.float32) + b2_ref[...]


def _mlp_t(x_t, W_comb, b_comb, W1, b1, W2, b2):
    return pl.pallas_call(
        _mlp_body,
        grid=(B // BLKC,),
        in_specs=[
            pl.BlockSpec((D_IN, BLKC), lambda i: (0, i)),
            pl.BlockSpec((D_OUT, D_IN), lambda i: (0, 0)),
            pl.BlockSpec((D_OUT, 1), lambda i: (0, 0)),
            pl.BlockSpec((D_HID, D_OUT), lambda i: (0, 0)),
            pl.BlockSpec((D_HID, 1), lambda i: (0, 0)),
            pl.BlockSpec((D_OUT, D_HID), lambda i: (0, 0)),
            pl.BlockSpec((D_OUT, 1), lambda i: (0, 0)),
        ],
        out_specs=pl.BlockSpec((D_OUT, BLKC), lambda i: (0, i)),
        out_shape=jax.ShapeDtypeStruct((D_OUT, B), jnp.float32),
    )(x_t, W_comb, b_comb.reshape(D_OUT, 1), W1, b1.reshape(D_HID, 1),
      W2, b2.reshape(D_OUT, 1))


def kernel(pert_indices, emb_table, W_comb, b_comb, W1, b1, W2, b2):
    idx = pert_indices.astype(jnp.int32)
    table_t = jnp.transpose(emb_table)
    x_t = _make_gather(emb_table.shape[0])(table_t, idx)
    out_t = _mlp_t(x_t, W_comb, b_comb, W1, b1, W2, b2)
    return jnp.transpose(out_t)


# speedup vs baseline: 1.0160x; 1.0160x over previous
import functools

import jax
import jax.numpy as jnp
from jax import lax
from jax.experimental import pallas as pl
from jax.experimental.pallas import tpu as pltpu
from jax.experimental.pallas import tpu_sc as plsc

B = 16384
D_IN = 64
D_HID = 128
D_OUT = 64
BLKC = 8192
CHUNK = 4096
L = 16
UNROLL = 8


@functools.lru_cache(maxsize=None)
def _make_gather(vocab):
    info = plsc.get_sparse_core_info()
    nc, ns = info.num_cores, info.num_subcores
    nw = nc * ns
    dims_per_w = D_IN // nw
    n_chunks = B // CHUNK

    mesh = plsc.VectorSubcoreMesh(core_axis_name="c", subcore_axis_name="s")

    @functools.partial(
        pl.kernel,
        mesh=mesh,
        out_type=jax.ShapeDtypeStruct((D_IN, B), jnp.float32),
        scratch_types=[
            pltpu.VMEM((B,), jnp.int32),
            pltpu.VMEM((vocab,), jnp.float32),
            pltpu.VMEM((CHUNK,), jnp.float32),
            pltpu.VMEM((CHUNK,), jnp.float32),
            pltpu.SemaphoreType.DMA,
            pltpu.SemaphoreType.DMA,
            pltpu.SemaphoreType.DMA,
            pltpu.SemaphoreType.DMA,
        ],
        compiler_params=pltpu.CompilerParams(needs_layout_passes=False),
    )
    def gather(table_t_hbm, idx_hbm, out_hbm, idx_v, col_v, out_a, out_b,
               sem_i, sem_c, sem_a, sem_b):
        wid = lax.axis_index("s") * nc + lax.axis_index("c")
        d0 = wid * dims_per_w
        idx_cp = pltpu.make_async_copy(idx_hbm, idx_v, sem_i)
        idx_cp.start()
        col_cp = pltpu.make_async_copy(table_t_hbm.at[d0], col_v, sem_c)
        col_cp.start()
        idx_cp.wait()
        col_cp.wait()
        outs = (out_a, out_b)
        sems = (sem_a, sem_b)
        for r in range(dims_per_w):
            d = d0 + r
            for chunk in range(n_chunks):
                out_v = outs[chunk % 2]
                sem_o = sems[chunk % 2]
                if r * n_chunks + chunk >= 2:
                    pltpu.make_async_copy(
                        out_v,
                        out_hbm.at[d0 + (r * n_chunks + chunk - 2) // n_chunks,
                                   pl.ds(((r * n_chunks + chunk - 2) % n_chunks)
                                         * CHUNK, CHUNK)],
                        sem_o).wait()

                @plsc.parallel_loop(0, CHUNK, step=L, unroll=UNROLL)
                def _(i, chunk=chunk, out_v=out_v):
                    iv = idx_v[pl.ds(chunk * CHUNK + i, L)]
                    out_v[pl.ds(i, L)] = plsc.load_gather(col_v, [iv])

                if r == 0 and chunk == n_chunks - 1 and dims_per_w > 1:
                    col_cp2 = pltpu.make_async_copy(
                        table_t_hbm.at[d0 + 1], col_v, sem_c)
                    col_cp2.start()
                pltpu.make_async_copy(
                    out_v, out_hbm.at[d, pl.ds(chunk * CHUNK, CHUNK)],
                    sem_o).start()
            if r == 0 and dims_per_w > 1:
                pltpu.make_async_copy(
                    table_t_hbm.at[d0 + 1], col_v, sem_c).wait()
        for chunk in (n_chunks - 2, n_chunks - 1):
            pltpu.make_async_copy(
                outs[chunk % 2],
                out_hbm.at[d0 + dims_per_w - 1, pl.ds(chunk * CHUNK, CHUNK)],
                sems[chunk % 2]).wait()

    return gather


def _mlp_body(x_ref, wc_ref, bc_ref, w1_ref, b1_ref, w2_ref, b2_ref, o_ref):
    dn = (((1,), (0,)), ((), ()))
    x = x_ref[...]
    c = lax.dot_general(wc_ref[...], x, dn,
                        preferred_element_type=jnp.float32) + bc_ref[...]
    h = jnp.maximum(
        lax.dot_general(w1_ref[...], c, dn,
                        preferred_element_type=jnp.float32) + b1_ref[...], 0.0)
    o_ref[...] = lax.dot_general(w2_ref[...], h, dn,
                                 preferred_element_type=jnp.float32) + b2_ref[...]


def _mlp_t(x_t, W_comb, b_comb, W1, b1, W2, b2):
    return pl.pallas_call(
        _mlp_body,
        grid=(B // BLKC,),
        in_specs=[
            pl.BlockSpec((D_IN, BLKC), lambda i: (0, i)),
            pl.BlockSpec((D_OUT, D_IN), lambda i: (0, 0)),
            pl.BlockSpec((D_OUT, 1), lambda i: (0, 0)),
            pl.BlockSpec((D_HID, D_OUT), lambda i: (0, 0)),
            pl.BlockSpec((D_HID, 1), lambda i: (0, 0)),
            pl.BlockSpec((D_OUT, D_HID), lambda i: (0, 0)),
            pl.BlockSpec((D_OUT, 1), lambda i: (0, 0)),
        ],
        out_specs=pl.BlockSpec((D_OUT, BLKC), lambda i: (0, i)),
        out_shape=jax.ShapeDtypeStruct((D_OUT, B), jnp.float32),
    )(x_t, W_comb, b_comb.reshape(D_OUT, 1), W1, b1.reshape(D_HID, 1),
      W2, b2.reshape(D_OUT, 1))


def kernel(pert_indices, emb_table, W_comb, b_comb, W1, b1, W2, b2):
    idx = pert_indices.astype(jnp.int32)
    table_t = jnp.transpose(emb_table)
    x_t = _make_gather(emb_table.shape[0])(table_t, idx)
    out_t = _mlp_t(x_t, W_comb, b_comb, W1, b1, W2, b2)
    return jnp.transpose(out_t)
